# Initial kernel scaffold; baseline (speedup 1.0000x reference)
import jax
import jax.numpy as jnp
from jax.experimental import pallas as pl


def kernel(user_h, item_h, i_time_enc, i_time_enc_k, u_time_enc, u_time_enc_k, gu_W1, gu_b1, gu_W2, gu_b2, gi_W1, gi_b1, gi_W2, gi_b2, u_neighbors, i_neighbors, u_times, i_times):
    raise NotImplementedError("write your pallas kernel here")


# speedup vs baseline: 2.0771x; 2.0771x over previous
import functools

import jax
import jax.numpy as jnp
from jax import lax
from jax.experimental import pallas as pl
from jax.experimental.pallas import tpu as pltpu
from jax.experimental.pallas import tpu_sc as plsc

_D = 128
_L = 50
_CH = 400


def _sc_gather(table, idx_flat, n_pad):
    info = plsc.get_sparse_core_info()
    nw = info.num_cores * info.num_subcores
    b_per_w = n_pad // nw
    n_ch = b_per_w // _CH
    mesh = plsc.VectorSubcoreMesh(core_axis_name="c", subcore_axis_name="s")

    @functools.partial(
        pl.kernel,
        out_type=jax.ShapeDtypeStruct((n_pad, _D), jnp.float32),
        mesh=mesh,
        scratch_types=[
            pltpu.VMEM((_CH,), jnp.int32),
            pltpu.VMEM((_CH, _D), jnp.float32),
            pltpu.SemaphoreType.DMA,
        ],
    )
    def k(table_hbm, idx_hbm, out_hbm, idx_v, rows_v, sem):
        wid = lax.axis_index("s") * info.num_cores + lax.axis_index("c")
        base = wid * b_per_w

        def body(c, carry):
            off = base + c * _CH
            pltpu.sync_copy(idx_hbm.at[pl.ds(off, _CH)], idx_v)
            pltpu.async_copy(table_hbm.at[idx_v], rows_v, sem).wait()
            pltpu.sync_copy(rows_v, out_hbm.at[pl.ds(off, _CH)])
            return carry

        lax.fori_loop(0, n_ch, body, 0)

    return k(table, idx_flat)


def _tc_body(mail_ref, dst_ref, t_ref, te_ref, tek_ref, w1_ref, b1_ref,
             w2_ref, b2_ref, out_ref):
    mail = mail_ref[...]
    dst = dst_ref[...]
    t = t_ref[...]
    scale = jnp.sqrt(jnp.float32(_D))

    tl = t[:, :, None]
    tm = t[:, None, :]
    li = lax.broadcasted_iota(jnp.int32, (1, _L, _L), 1)
    mi = lax.broadcasted_iota(jnp.int32, (1, _L, _L), 2)
    cmp = (tm < tl) | ((tm == tl) & (mi < li))
    order = jnp.sum(cmp.astype(jnp.int32), axis=2)
    re_order = (_L - 1) - order

    tedot = lax.dot_general(dst, te_ref[...],
                            (((1,), (1,)), ((), ())))
    ji = lax.broadcasted_iota(jnp.int32, (1, _L, _L), 2)
    ohf = (re_order[:, :, None] == ji).astype(jnp.float32)
    e_te = jnp.sum(ohf * tedot[:, None, :], axis=2)
    s = jnp.sum(mail * dst[:, None, :], axis=2)
    e = (e_te + s) / scale
    e = e - jnp.max(e, axis=1, keepdims=True)
    ex = jnp.exp(e)
    alpha = ex / jnp.sum(ex, axis=1, keepdims=True)

    beta = jnp.sum(ohf * alpha[:, :, None], axis=1)
    tek_term = jnp.dot(beta, tek_ref[...])

    tmax = jnp.max(t, axis=1, keepdims=True)
    l1 = lax.broadcasted_iota(jnp.int32, (1, _L), 1)
    lsel = jnp.min(jnp.where(t == tmax, l1, _L), axis=1, keepdims=True)
    last_oh = (l1 == lsel).astype(jnp.float32)
    last_em = jnp.sum(last_oh[:, :, None] * mail, axis=1)
    e1 = jnp.sum(mail * last_em[:, None, :], axis=2) / scale
    e1 = e1 - jnp.max(e1, axis=1, keepdims=True)
    ex1 = jnp.exp(e1)
    alpha1 = ex1 / jnp.sum(ex1, axis=1, keepdims=True)

    w = alpha + alpha1
    hmail = jnp.sum(w[:, :, None] * mail, axis=1)
    h = hmail + tek_term

    z = jnp.maximum(jnp.dot(h, w1_ref[...]) + b1_ref[...], 0.0)
    z = jnp.dot(z, w2_ref[...]) + b2_ref[...] + dst
    out_ref[...] = jnp.where(z > 0.0, z, jnp.exp(z) - 1.0)


def _tc_side(mail, dst_h, times, te, tek, w1, b1, w2, b2, block_b=80,
             interpret=False):
    n = dst_h.shape[0]
    grid = n // block_b
    return pl.pallas_call(
        _tc_body,
        grid=(grid,),
        in_specs=[
            pl.BlockSpec((block_b, _L, _D), lambda i: (i, 0, 0)),
            pl.BlockSpec((block_b, _D), lambda i: (i, 0)),
            pl.BlockSpec((block_b, _L), lambda i: (i, 0)),
            pl.BlockSpec((_L, _D), lambda i: (0, 0)),
            pl.BlockSpec((_L, _D), lambda i: (0, 0)),
            pl.BlockSpec((_D, _D), lambda i: (0, 0)),
            pl.BlockSpec((1, _D), lambda i: (0, 0)),
            pl.BlockSpec((_D, _D), lambda i: (0, 0)),
            pl.BlockSpec((1, _D), lambda i: (0, 0)),
        ],
        out_specs=pl.BlockSpec((block_b, _D), lambda i: (i, 0)),
        out_shape=jax.ShapeDtypeStruct((n, _D), jnp.float32),
        interpret=interpret,
    )(mail, dst_h, times, te, tek, w1, b1.reshape(1, _D), w2,
      b2.reshape(1, _D))


def kernel(user_h, item_h, i_time_enc, i_time_enc_k, u_time_enc,
           u_time_enc_k, gu_W1, gu_b1, gu_W2, gu_b2, gi_W1, gi_b1, gi_W2,
           gi_b2, u_neighbors, i_neighbors, u_times, i_times):
    n_item, l = i_neighbors.shape
    n_user = u_neighbors.shape[0]
    info = plsc.get_sparse_core_info()
    nw = info.num_cores * info.num_subcores
    step = nw * _CH

    def gather_side(table, idx):
        n_rows = idx.shape[0] * l
        n_pad = ((n_rows + step - 1) // step) * step
        idx_flat = jnp.concatenate(
            [idx.reshape(-1),
             jnp.zeros((n_pad - n_rows,), dtype=jnp.int32)])
        rows = _sc_gather(table, idx_flat, n_pad)
        return rows[:n_rows].reshape(idx.shape[0], l, _D)

    mail_for_item = gather_side(user_h, i_neighbors)
    mail_for_user = gather_side(item_h, u_neighbors)

    item_new = _tc_side(mail_for_item, item_h, i_times, i_time_enc,
                        i_time_enc_k, gi_W1, gi_b1, gi_W2, gi_b2)
    user_new = _tc_side(mail_for_user, user_h, u_times, u_time_enc,
                        u_time_enc_k, gu_W1, gu_b1, gu_W2, gu_b2)
    return (user_new, item_new)


# speedup vs baseline: 2.1381x; 1.0294x over previous
import functools

import jax
import jax.numpy as jnp
from jax import lax
from jax.experimental import pallas as pl
from jax.experimental.pallas import tpu as pltpu
from jax.experimental.pallas import tpu_sc as plsc

_D = 128
_L = 50
_CH = 400


def _sc_gather(table, idx_flat, n_pad):
    info = plsc.get_sparse_core_info()
    nw = info.num_cores * info.num_subcores
    b_per_w = n_pad // nw
    n_ch = b_per_w // _CH
    assert n_ch >= 4 and n_ch % 2 == 0
    mesh = plsc.VectorSubcoreMesh(core_axis_name="c", subcore_axis_name="s")

    @functools.partial(
        pl.kernel,
        out_type=jax.ShapeDtypeStruct((n_pad, _D), jnp.float32),
        mesh=mesh,
        scratch_types=[
            pltpu.VMEM((b_per_w,), jnp.int32),
            pltpu.VMEM((2, _CH, _D), jnp.float32),
            pltpu.SemaphoreType.DMA,
            pltpu.SemaphoreType.DMA,
            pltpu.SemaphoreType.DMA,
            pltpu.SemaphoreType.DMA,
        ],
    )
    def k(table_hbm, idx_hbm, out_hbm, idx_v, rows_v, sg0, sg1, sw0, sw1):
        wid = lax.axis_index("s") * info.num_cores + lax.axis_index("c")
        base = wid * b_per_w
        pltpu.sync_copy(idx_hbm.at[pl.ds(base, b_per_w)], idx_v)

        def g_copy(c, buf, sem):
            return pltpu.make_async_copy(
                table_hbm.at[idx_v.at[pl.ds(c * _CH, _CH)]],
                rows_v.at[buf], sem)

        def w_copy(c, buf, sem):
            return pltpu.make_async_copy(
                rows_v.at[buf], out_hbm.at[pl.ds(base + c * _CH, _CH)], sem)

        g_copy(0, 0, sg0).start()
        g_copy(0, 0, sg0).wait()
        w_copy(0, 0, sw0).start()
        g_copy(1, 1, sg1).start()

        def body(i, carry):
            c = 1 + 2 * i
            g_copy(c, 1, sg1).wait()
            w_copy(c, 1, sw1).start()
            w_copy(c - 1, 0, sw0).wait()
            g_copy(c + 1, 0, sg0).start()
            g_copy(c + 1, 0, sg0).wait()
            w_copy(c + 1, 0, sw0).start()
            w_copy(c, 1, sw1).wait()
            g_copy(c + 2, 1, sg1).start()
            return carry

        lax.fori_loop(0, (n_ch - 2) // 2, body, 0)

        c_last = n_ch - 1
        g_copy(c_last, 1, sg1).wait()
        w_copy(c_last, 1, sw1).start()
        w_copy(c_last - 1, 0, sw0).wait()
        w_copy(c_last, 1, sw1).wait()

    return k(table, idx_flat)


def _tc_body(mail_ref, dst_ref, t_ref, te_ref, tek_ref, w1_ref, b1_ref,
             w2_ref, b2_ref, out_ref):
    mail = mail_ref[...]
    dst = dst_ref[...]
    t = t_ref[...]
    scale = jnp.sqrt(jnp.float32(_D))

    tl = t[:, :, None]
    tm = t[:, None, :]
    li = lax.broadcasted_iota(jnp.int32, (1, _L, _L), 1)
    mi = lax.broadcasted_iota(jnp.int32, (1, _L, _L), 2)
    cmp = (tm < tl) | ((tm == tl) & (mi < li))
    order = jnp.sum(cmp.astype(jnp.int32), axis=2)
    re_order = (_L - 1) - order

    tedot = lax.dot_general(dst, te_ref[...],
                            (((1,), (1,)), ((), ())))
    ji = lax.broadcasted_iota(jnp.int32, (1, _L, _L), 2)
    ohf = (re_order[:, :, None] == ji).astype(jnp.float32)
    e_te = jnp.sum(ohf * tedot[:, None, :], axis=2)
    s = jnp.sum(mail * dst[:, None, :], axis=2)
    e = (e_te + s) / scale
    e = e - jnp.max(e, axis=1, keepdims=True)
    ex = jnp.exp(e)
    alpha = ex / jnp.sum(ex, axis=1, keepdims=True)

    beta = jnp.sum(ohf * alpha[:, :, None], axis=1)
    tek_term = jnp.dot(beta, tek_ref[...])

    tmax = jnp.max(t, axis=1, keepdims=True)
    l1 = lax.broadcasted_iota(jnp.int32, (1, _L), 1)
    lsel = jnp.min(jnp.where(t == tmax, l1, _L), axis=1, keepdims=True)
    last_oh = (l1 == lsel).astype(jnp.float32)
    last_em = jnp.sum(last_oh[:, :, None] * mail, axis=1)
    e1 = jnp.sum(mail * last_em[:, None, :], axis=2) / scale
    e1 = e1 - jnp.max(e1, axis=1, keepdims=True)
    ex1 = jnp.exp(e1)
    alpha1 = ex1 / jnp.sum(ex1, axis=1, keepdims=True)

    w = alpha + alpha1
    hmail = jnp.sum(w[:, :, None] * mail, axis=1)
    h = hmail + tek_term

    z = jnp.maximum(jnp.dot(h, w1_ref[...]) + b1_ref[...], 0.0)
    z = jnp.dot(z, w2_ref[...]) + b2_ref[...] + dst
    out_ref[...] = jnp.where(z > 0.0, z, jnp.exp(z) - 1.0)


def _tc_side(mail, dst_h, times, te, tek, w1, b1, w2, b2, block_b=80,
             interpret=False):
    n = dst_h.shape[0]
    grid = n // block_b
    return pl.pallas_call(
        _tc_body,
        grid=(grid,),
        in_specs=[
            pl.BlockSpec((block_b, _L, _D), lambda i: (i, 0, 0)),
            pl.BlockSpec((block_b, _D), lambda i: (i, 0)),
            pl.BlockSpec((block_b, _L), lambda i: (i, 0)),
            pl.BlockSpec((_L, _D), lambda i: (0, 0)),
            pl.BlockSpec((_L, _D), lambda i: (0, 0)),
            pl.BlockSpec((_D, _D), lambda i: (0, 0)),
            pl.BlockSpec((1, _D), lambda i: (0, 0)),
            pl.BlockSpec((_D, _D), lambda i: (0, 0)),
            pl.BlockSpec((1, _D), lambda i: (0, 0)),
        ],
        out_specs=pl.BlockSpec((block_b, _D), lambda i: (i, 0)),
        out_shape=jax.ShapeDtypeStruct((n, _D), jnp.float32),
        interpret=interpret,
    )(mail, dst_h, times, te, tek, w1, b1.reshape(1, _D), w2,
      b2.reshape(1, _D))


def kernel(user_h, item_h, i_time_enc, i_time_enc_k, u_time_enc,
           u_time_enc_k, gu_W1, gu_b1, gu_W2, gu_b2, gi_W1, gi_b1, gi_W2,
           gi_b2, u_neighbors, i_neighbors, u_times, i_times):
    n_item, l = i_neighbors.shape
    n_user = u_neighbors.shape[0]
    info = plsc.get_sparse_core_info()
    nw = info.num_cores * info.num_subcores
    step = nw * _CH

    def gather_side(table, idx):
        n_rows = idx.shape[0] * l
        n_pad = ((n_rows + step - 1) // step) * step
        idx_flat = jnp.concatenate(
            [idx.reshape(-1),
             jnp.zeros((n_pad - n_rows,), dtype=jnp.int32)])
        rows = _sc_gather(table, idx_flat, n_pad)
        return rows[:n_rows].reshape(idx.shape[0], l, _D)

    mail_for_item = gather_side(user_h, i_neighbors)
    mail_for_user = gather_side(item_h, u_neighbors)

    item_new = _tc_side(mail_for_item, item_h, i_times, i_time_enc,
                        i_time_enc_k, gi_W1, gi_b1, gi_W2, gi_b2)
    user_new = _tc_side(mail_for_user, user_h, u_times, u_time_enc,
                        u_time_enc_k, gu_W1, gu_b1, gu_W2, gu_b2)
    return (user_new, item_new)


# speedup vs baseline: 2.2776x; 1.0652x over previous
import functools

import jax
import jax.numpy as jnp
from jax import lax
from jax.experimental import pallas as pl
from jax.experimental.pallas import tpu as pltpu
from jax.experimental.pallas import tpu_sc as plsc

_D = 128
_L = 50
_CH = 400


def _sc_gather(table, idx_flat, n_pad):
    info = plsc.get_sparse_core_info()
    nw = info.num_cores * info.num_subcores
    b_per_w = n_pad // nw
    n_ch = b_per_w // _CH
    assert n_ch >= 4 and n_ch % 2 == 0
    mesh = plsc.VectorSubcoreMesh(core_axis_name="c", subcore_axis_name="s")

    @functools.partial(
        pl.kernel,
        out_type=jax.ShapeDtypeStruct((n_pad, _D), jnp.float32),
        mesh=mesh,
        scratch_types=[
            pltpu.VMEM((b_per_w,), jnp.int32),
            pltpu.VMEM((2, _CH, _D), jnp.float32),
            pltpu.SemaphoreType.DMA,
            pltpu.SemaphoreType.DMA,
            pltpu.SemaphoreType.DMA,
            pltpu.SemaphoreType.DMA,
        ],
    )
    def k(table_hbm, idx_hbm, out_hbm, idx_v, rows_v, sg0, sg1, sw0, sw1):
        wid = lax.axis_index("s") * info.num_cores + lax.axis_index("c")
        base = wid * b_per_w
        pltpu.sync_copy(idx_hbm.at[pl.ds(base, b_per_w)], idx_v)

        def g_copy(c, buf, sem):
            return pltpu.make_async_copy(
                table_hbm.at[idx_v.at[pl.ds(c * _CH, _CH)]],
                rows_v.at[buf], sem)

        def w_copy(c, buf, sem):
            return pltpu.make_async_copy(
                rows_v.at[buf], out_hbm.at[pl.ds(base + c * _CH, _CH)], sem)

        g_copy(0, 0, sg0).start()
        g_copy(0, 0, sg0).wait()
        w_copy(0, 0, sw0).start()
        g_copy(1, 1, sg1).start()

        def body(i, carry):
            c = 1 + 2 * i
            g_copy(c, 1, sg1).wait()
            w_copy(c, 1, sw1).start()
            w_copy(c - 1, 0, sw0).wait()
            g_copy(c + 1, 0, sg0).start()
            g_copy(c + 1, 0, sg0).wait()
            w_copy(c + 1, 0, sw0).start()
            w_copy(c, 1, sw1).wait()
            g_copy(c + 2, 1, sg1).start()
            return carry

        lax.fori_loop(0, (n_ch - 2) // 2, body, 0)

        c_last = n_ch - 1
        g_copy(c_last, 1, sg1).wait()
        w_copy(c_last, 1, sw1).start()
        w_copy(c_last - 1, 0, sw0).wait()
        w_copy(c_last, 1, sw1).wait()

    return k(table, idx_flat)


def _tc_body(mail_ref, dst_ref, t_ref, te_ref, tek_ref, w1_ref, b1_ref,
             w2_ref, b2_ref, out_ref):
    mail = mail_ref[...]
    dst = dst_ref[...]
    t = t_ref[...]
    inv_scale = float(1.0 / (128.0 ** 0.5))

    tl = t[:, None, :]
    tm = t[:, :, None]
    li = lax.broadcasted_iota(jnp.int32, (1, _L, _L), 2)
    mi = lax.broadcasted_iota(jnp.int32, (1, _L, _L), 1)
    cmp = (tm < tl) | ((tm == tl) & (mi < li))
    order = jnp.sum(cmp.astype(jnp.int32), axis=1)
    re_order = (_L - 1) - order

    tedot = lax.dot_general(dst, te_ref[...],
                            (((1,), (1,)), ((), ())))
    ji = lax.broadcasted_iota(jnp.int32, (1, _L, _L), 2)
    ohf = (re_order[:, :, None] == ji).astype(jnp.float32)
    e_te = jnp.sum(ohf * tedot[:, None, :], axis=2)
    s = jnp.sum(mail * dst[:, None, :], axis=2)
    e = (e_te + s) * inv_scale
    e = e - jnp.max(e, axis=1, keepdims=True)
    ex = jnp.exp(e)
    alpha = ex * (1.0 / jnp.sum(ex, axis=1, keepdims=True))

    beta = jnp.sum(ohf * alpha[:, :, None], axis=1)
    tek_term = jnp.dot(beta, tek_ref[...])

    tmax = jnp.max(t, axis=1, keepdims=True)
    l1 = lax.broadcasted_iota(jnp.int32, (1, _L), 1)
    lsel = jnp.min(jnp.where(t == tmax, l1, _L), axis=1, keepdims=True)
    last_oh = (l1 == lsel).astype(jnp.float32)
    last_em = jnp.sum(last_oh[:, :, None] * mail, axis=1)
    e1 = jnp.sum(mail * last_em[:, None, :], axis=2) * inv_scale
    e1 = e1 - jnp.max(e1, axis=1, keepdims=True)
    ex1 = jnp.exp(e1)
    alpha1 = ex1 * (1.0 / jnp.sum(ex1, axis=1, keepdims=True))

    w = alpha + alpha1
    hmail = jnp.sum(w[:, :, None] * mail, axis=1)
    h = hmail + tek_term

    z = jnp.maximum(jnp.dot(h, w1_ref[...]) + b1_ref[...], 0.0)
    z = jnp.dot(z, w2_ref[...]) + b2_ref[...] + dst
    out_ref[...] = jnp.where(z > 0.0, z, jnp.exp(z) - 1.0)


def _tc_side(mail, dst_h, times, te, tek, w1, b1, w2, b2, block_b=200,
             interpret=False):
    n = dst_h.shape[0]
    grid = n // block_b
    return pl.pallas_call(
        _tc_body,
        grid=(grid,),
        in_specs=[
            pl.BlockSpec((block_b, _L, _D), lambda i: (i, 0, 0)),
            pl.BlockSpec((block_b, _D), lambda i: (i, 0)),
            pl.BlockSpec((block_b, _L), lambda i: (i, 0)),
            pl.BlockSpec((_L, _D), lambda i: (0, 0)),
            pl.BlockSpec((_L, _D), lambda i: (0, 0)),
            pl.BlockSpec((_D, _D), lambda i: (0, 0)),
            pl.BlockSpec((1, _D), lambda i: (0, 0)),
            pl.BlockSpec((_D, _D), lambda i: (0, 0)),
            pl.BlockSpec((1, _D), lambda i: (0, 0)),
        ],
        out_specs=pl.BlockSpec((block_b, _D), lambda i: (i, 0)),
        out_shape=jax.ShapeDtypeStruct((n, _D), jnp.float32),
        interpret=interpret,
    )(mail, dst_h, times, te, tek, w1, b1.reshape(1, _D), w2,
      b2.reshape(1, _D))


def kernel(user_h, item_h, i_time_enc, i_time_enc_k, u_time_enc,
           u_time_enc_k, gu_W1, gu_b1, gu_W2, gu_b2, gi_W1, gi_b1, gi_W2,
           gi_b2, u_neighbors, i_neighbors, u_times, i_times):
    n_item, l = i_neighbors.shape
    n_user = u_neighbors.shape[0]
    info = plsc.get_sparse_core_info()
    nw = info.num_cores * info.num_subcores
    step = nw * _CH

    def gather_side(table, idx):
        n_rows = idx.shape[0] * l
        n_pad = ((n_rows + step - 1) // step) * step
        idx_flat = jnp.concatenate(
            [idx.reshape(-1),
             jnp.zeros((n_pad - n_rows,), dtype=jnp.int32)])
        rows = _sc_gather(table, idx_flat, n_pad)
        return rows[:n_rows].reshape(idx.shape[0], l, _D)

    mail_for_item = gather_side(user_h, i_neighbors)
    mail_for_user = gather_side(item_h, u_neighbors)

    item_new = _tc_side(mail_for_item, item_h, i_times, i_time_enc,
                        i_time_enc_k, gi_W1, gi_b1, gi_W2, gi_b2)
    user_new = _tc_side(mail_for_user, user_h, u_times, u_time_enc,
                        u_time_enc_k, gu_W1, gu_b1, gu_W2, gu_b2)
    return (user_new, item_new)


# speedup vs baseline: 2.4867x; 1.0918x over previous
import functools

import jax
import jax.numpy as jnp
from jax import lax
from jax.experimental import pallas as pl
from jax.experimental.pallas import tpu as pltpu
from jax.experimental.pallas import tpu_sc as plsc

_D = 128
_L = 50
_CH = 128


def _sc_gather(table, idx_flat, n_pad):
    info = plsc.get_sparse_core_info()
    nw = info.num_cores * info.num_subcores
    b_per_w = n_pad // nw
    n_ch = b_per_w // _CH
    assert n_ch >= 4 and n_ch % 2 == 0
    mesh = plsc.VectorSubcoreMesh(core_axis_name="c", subcore_axis_name="s")

    @functools.partial(
        pl.kernel,
        out_type=jax.ShapeDtypeStruct((n_pad, _D), jnp.float32),
        mesh=mesh,
        scratch_types=[
            pltpu.VMEM((b_per_w,), jnp.int32),
            pltpu.VMEM((2, _CH, _D), jnp.float32),
            pltpu.SemaphoreType.DMA,
            pltpu.SemaphoreType.DMA,
            pltpu.SemaphoreType.DMA,
            pltpu.SemaphoreType.DMA,
        ],
    )
    def k(table_hbm, idx_hbm, out_hbm, idx_v, rows_v, sg0, sg1, sw0, sw1):
        wid = lax.axis_index("s") * info.num_cores + lax.axis_index("c")
        base = wid * b_per_w
        pltpu.sync_copy(idx_hbm.at[pl.ds(base, b_per_w)], idx_v)

        def g_copy(c, buf, sem):
            return pltpu.make_async_copy(
                table_hbm.at[idx_v.at[pl.ds(c * _CH, _CH)]],
                rows_v.at[buf], sem)

        def w_copy(c, buf, sem):
            return pltpu.make_async_copy(
                rows_v.at[buf], out_hbm.at[pl.ds(base + c * _CH, _CH)], sem)

        g_copy(0, 0, sg0).start()
        g_copy(0, 0, sg0).wait()
        w_copy(0, 0, sw0).start()
        g_copy(1, 1, sg1).start()

        def body(i, carry):
            c = 1 + 2 * i
            g_copy(c, 1, sg1).wait()
            w_copy(c, 1, sw1).start()
            w_copy(c - 1, 0, sw0).wait()
            g_copy(c + 1, 0, sg0).start()
            g_copy(c + 1, 0, sg0).wait()
            w_copy(c + 1, 0, sw0).start()
            w_copy(c, 1, sw1).wait()
            g_copy(c + 2, 1, sg1).start()
            return carry

        lax.fori_loop(0, (n_ch - 2) // 2, body, 0)

        c_last = n_ch - 1
        g_copy(c_last, 1, sg1).wait()
        w_copy(c_last, 1, sw1).start()
        w_copy(c_last - 1, 0, sw0).wait()
        w_copy(c_last, 1, sw1).wait()

    return k(table, idx_flat)


def _tc_body(mail_ref, dst_ref, t_ref, te_ref, tek_ref, w1_ref, b1_ref,
             w2_ref, b2_ref, out_ref):
    mail = mail_ref[...]
    dst = dst_ref[...]
    t = t_ref[...]
    inv_scale = float(1.0 / (128.0 ** 0.5))

    tl = t[:, None, :]
    tm = t[:, :, None]
    li = lax.broadcasted_iota(jnp.int32, (1, _L, _L), 2)
    mi = lax.broadcasted_iota(jnp.int32, (1, _L, _L), 1)
    cmp = (tm < tl) | ((tm == tl) & (mi < li))
    order = jnp.sum(cmp.astype(jnp.int32), axis=1)
    re_order = (_L - 1) - order

    tedot = lax.dot_general(dst, te_ref[...],
                            (((1,), (1,)), ((), ())))
    ji = lax.broadcasted_iota(jnp.int32, (1, _L, _L), 2)
    ohf = (re_order[:, :, None] == ji).astype(jnp.float32)
    e_te = jnp.sum(ohf * tedot[:, None, :], axis=2)
    s = jnp.sum(mail * dst[:, None, :], axis=2)
    e = (e_te + s) * inv_scale
    e = e - jnp.max(e, axis=1, keepdims=True)
    ex = jnp.exp(e)
    alpha = ex * (1.0 / jnp.sum(ex, axis=1, keepdims=True))

    beta = jnp.sum(ohf * alpha[:, :, None], axis=1)
    tek_term = jnp.dot(beta, tek_ref[...])

    tmax = jnp.max(t, axis=1, keepdims=True)
    l1 = lax.broadcasted_iota(jnp.int32, (1, _L), 1)
    lsel = jnp.min(jnp.where(t == tmax, l1, _L), axis=1, keepdims=True)
    last_oh = (l1 == lsel).astype(jnp.float32)
    last_em = jnp.sum(last_oh[:, :, None] * mail, axis=1)
    e1 = jnp.sum(mail * last_em[:, None, :], axis=2) * inv_scale
    e1 = e1 - jnp.max(e1, axis=1, keepdims=True)
    ex1 = jnp.exp(e1)
    alpha1 = ex1 * (1.0 / jnp.sum(ex1, axis=1, keepdims=True))

    w = alpha + alpha1
    hmail = jnp.sum(w[:, :, None] * mail, axis=1)
    h = hmail + tek_term

    z = jnp.maximum(jnp.dot(h, w1_ref[...]) + b1_ref[...], 0.0)
    z = jnp.dot(z, w2_ref[...]) + b2_ref[...] + dst
    out_ref[...] = jnp.where(z > 0.0, z, jnp.exp(z) - 1.0)


def _tc_side(mail, dst_h, times, te, tek, w1, b1, w2, b2, block_b=200,
             interpret=False):
    n = dst_h.shape[0]
    grid = n // block_b
    return pl.pallas_call(
        _tc_body,
        grid=(grid,),
        in_specs=[
            pl.BlockSpec((block_b, _L, _D), lambda i: (i, 0, 0)),
            pl.BlockSpec((block_b, _D), lambda i: (i, 0)),
            pl.BlockSpec((block_b, _L), lambda i: (i, 0)),
            pl.BlockSpec((_L, _D), lambda i: (0, 0)),
            pl.BlockSpec((_L, _D), lambda i: (0, 0)),
            pl.BlockSpec((_D, _D), lambda i: (0, 0)),
            pl.BlockSpec((1, _D), lambda i: (0, 0)),
            pl.BlockSpec((_D, _D), lambda i: (0, 0)),
            pl.BlockSpec((1, _D), lambda i: (0, 0)),
        ],
        out_specs=pl.BlockSpec((block_b, _D), lambda i: (i, 0)),
        out_shape=jax.ShapeDtypeStruct((n, _D), jnp.float32),
        interpret=interpret,
    )(mail, dst_h, times, te, tek, w1, b1.reshape(1, _D), w2,
      b2.reshape(1, _D))


def kernel(user_h, item_h, i_time_enc, i_time_enc_k, u_time_enc,
           u_time_enc_k, gu_W1, gu_b1, gu_W2, gu_b2, gi_W1, gi_b1, gi_W2,
           gi_b2, u_neighbors, i_neighbors, u_times, i_times):
    n_item, l = i_neighbors.shape
    n_user = u_neighbors.shape[0]
    info = plsc.get_sparse_core_info()
    nw = info.num_cores * info.num_subcores
    step = nw * _CH

    def gather_side(table, idx):
        n_rows = idx.shape[0] * l
        step2 = 2 * step
        n_pad = ((n_rows + step2 - 1) // step2) * step2
        idx_flat = jnp.concatenate(
            [idx.reshape(-1),
             jnp.zeros((n_pad - n_rows,), dtype=jnp.int32)])
        rows = _sc_gather(table, idx_flat, n_pad)
        return rows[:n_rows].reshape(idx.shape[0], l, _D)

    mail_for_item = gather_side(user_h, i_neighbors)
    mail_for_user = gather_side(item_h, u_neighbors)

    item_new = _tc_side(mail_for_item, item_h, i_times, i_time_enc,
                        i_time_enc_k, gi_W1, gi_b1, gi_W2, gi_b2)
    user_new = _tc_side(mail_for_user, user_h, u_times, u_time_enc,
                        u_time_enc_k, gu_W1, gu_b1, gu_W2, gu_b2)
    return (user_new, item_new)


# speedup vs baseline: 2.5518x; 1.0262x over previous
import functools

import jax
import jax.numpy as jnp
from jax import lax
from jax.experimental import pallas as pl
from jax.experimental.pallas import tpu as pltpu
from jax.experimental.pallas import tpu_sc as plsc

_D = 128
_L = 50
_CH = 128


def _sc_gather(table, idx_flat, n_pad):
    info = plsc.get_sparse_core_info()
    nw = info.num_cores * info.num_subcores
    b_per_w = n_pad // nw
    n_ch = b_per_w // _CH
    assert n_ch >= 8 and n_ch % 4 == 0
    mesh = plsc.VectorSubcoreMesh(core_axis_name="c", subcore_axis_name="s")

    @functools.partial(
        pl.kernel,
        out_type=jax.ShapeDtypeStruct((n_pad, _D), jnp.float32),
        mesh=mesh,
        scratch_types=[
            pltpu.VMEM((b_per_w,), jnp.int32),
            pltpu.VMEM((4, _CH, _D), jnp.float32),
            [pltpu.SemaphoreType.DMA] * 4,
            [pltpu.SemaphoreType.DMA] * 4,
        ],
    )
    def k(table_hbm, idx_hbm, out_hbm, idx_v, rows_v, sg, sw):
        wid = lax.axis_index("s") * info.num_cores + lax.axis_index("c")
        base = wid * b_per_w
        pltpu.sync_copy(idx_hbm.at[pl.ds(base, b_per_w)], idx_v)

        def g_copy(c, buf):
            return pltpu.make_async_copy(
                table_hbm.at[idx_v.at[pl.ds(c * _CH, _CH)]],
                rows_v.at[buf], sg[buf])

        def w_copy(c, buf):
            return pltpu.make_async_copy(
                rows_v.at[buf], out_hbm.at[pl.ds(base + c * _CH, _CH)],
                sw[buf])

        g_copy(0, 0).start()
        g_copy(1, 1).start()
        g_copy(2, 2).start()
        g_copy(0, 0).wait()
        w_copy(0, 0).start()
        g_copy(3, 3).start()

        def body(q, carry):
            for kq in range(4):
                c = 1 + 4 * q + kq
                b = (1 + kq) % 4
                pb = kq % 4
                g_copy(c, b).wait()
                w_copy(c, b).start()
                w_copy(c - 1, pb).wait()
                g_copy(c + 3, pb).start()
            return carry

        lax.fori_loop(0, (n_ch - 4) // 4, body, 0)

        for c in (n_ch - 3, n_ch - 2, n_ch - 1):
            b = c % 4
            g_copy(c, b).wait()
            w_copy(c, b).start()
        for c in (n_ch - 4, n_ch - 3, n_ch - 2, n_ch - 1):
            w_copy(c, c % 4).wait()

    return k(table, idx_flat)


def _tc_body(mail_ref, dst_ref, t_ref, te_ref, tek_ref, w1_ref, b1_ref,
             w2_ref, b2_ref, out_ref):
    mail = mail_ref[...]
    dst = dst_ref[...]
    t = t_ref[...]
    inv_scale = float(1.0 / (128.0 ** 0.5))

    tl = t[:, None, :]
    tm = t[:, :, None]
    li = lax.broadcasted_iota(jnp.int32, (1, _L, _L), 2)
    mi = lax.broadcasted_iota(jnp.int32, (1, _L, _L), 1)
    cmp = (tm < tl) | ((tm == tl) & (mi < li))
    order = jnp.sum(cmp.astype(jnp.int32), axis=1)
    re_order = (_L - 1) - order

    tedot = lax.dot_general(dst, te_ref[...],
                            (((1,), (1,)), ((), ())))
    ji = lax.broadcasted_iota(jnp.int32, (1, _L, _L), 2)
    ohf = (re_order[:, :, None] == ji).astype(jnp.float32)
    e_te = jnp.sum(ohf * tedot[:, None, :], axis=2)
    s = jnp.sum(mail * dst[:, None, :], axis=2)
    e = (e_te + s) * inv_scale
    e = e - jnp.max(e, axis=1, keepdims=True)
    ex = jnp.exp(e)
    alpha = ex * (1.0 / jnp.sum(ex, axis=1, keepdims=True))

    beta = jnp.sum(ohf * alpha[:, :, None], axis=1)
    tek_term = jnp.dot(beta, tek_ref[...])

    tmax = jnp.max(t, axis=1, keepdims=True)
    l1 = lax.broadcasted_iota(jnp.int32, (1, _L), 1)
    lsel = jnp.min(jnp.where(t == tmax, l1, _L), axis=1, keepdims=True)
    last_oh = (l1 == lsel).astype(jnp.float32)
    last_em = jnp.sum(last_oh[:, :, None] * mail, axis=1)
    e1 = jnp.sum(mail * last_em[:, None, :], axis=2) * inv_scale
    e1 = e1 - jnp.max(e1, axis=1, keepdims=True)
    ex1 = jnp.exp(e1)
    alpha1 = ex1 * (1.0 / jnp.sum(ex1, axis=1, keepdims=True))

    w = alpha + alpha1
    hmail = jnp.sum(w[:, :, None] * mail, axis=1)
    h = hmail + tek_term

    z = jnp.maximum(jnp.dot(h, w1_ref[...]) + b1_ref[...], 0.0)
    z = jnp.dot(z, w2_ref[...]) + b2_ref[...] + dst
    out_ref[...] = jnp.where(z > 0.0, z, jnp.exp(z) - 1.0)


def _tc_side(mail, dst_h, times, te, tek, w1, b1, w2, b2, block_b=200,
             interpret=False):
    n = dst_h.shape[0]
    grid = n // block_b
    return pl.pallas_call(
        _tc_body,
        grid=(grid,),
        in_specs=[
            pl.BlockSpec((block_b, _L, _D), lambda i: (i, 0, 0)),
            pl.BlockSpec((block_b, _D), lambda i: (i, 0)),
            pl.BlockSpec((block_b, _L), lambda i: (i, 0)),
            pl.BlockSpec((_L, _D), lambda i: (0, 0)),
            pl.BlockSpec((_L, _D), lambda i: (0, 0)),
            pl.BlockSpec((_D, _D), lambda i: (0, 0)),
            pl.BlockSpec((1, _D), lambda i: (0, 0)),
            pl.BlockSpec((_D, _D), lambda i: (0, 0)),
            pl.BlockSpec((1, _D), lambda i: (0, 0)),
        ],
        out_specs=pl.BlockSpec((block_b, _D), lambda i: (i, 0)),
        out_shape=jax.ShapeDtypeStruct((n, _D), jnp.float32),
        interpret=interpret,
    )(mail, dst_h, times, te, tek, w1, b1.reshape(1, _D), w2,
      b2.reshape(1, _D))


def kernel(user_h, item_h, i_time_enc, i_time_enc_k, u_time_enc,
           u_time_enc_k, gu_W1, gu_b1, gu_W2, gu_b2, gi_W1, gi_b1, gi_W2,
           gi_b2, u_neighbors, i_neighbors, u_times, i_times):
    n_item, l = i_neighbors.shape
    n_user = u_neighbors.shape[0]
    info = plsc.get_sparse_core_info()
    nw = info.num_cores * info.num_subcores
    step = nw * _CH

    def gather_side(table, idx):
        n_rows = idx.shape[0] * l
        step2 = 4 * step
        n_pad = ((n_rows + step2 - 1) // step2) * step2
        idx_flat = jnp.concatenate(
            [idx.reshape(-1),
             jnp.zeros((n_pad - n_rows,), dtype=jnp.int32)])
        rows = _sc_gather(table, idx_flat, n_pad)
        return rows[:n_rows].reshape(idx.shape[0], l, _D)

    mail_for_item = gather_side(user_h, i_neighbors)
    mail_for_user = gather_side(item_h, u_neighbors)

    item_new = _tc_side(mail_for_item, item_h, i_times, i_time_enc,
                        i_time_enc_k, gi_W1, gi_b1, gi_W2, gi_b2)
    user_new = _tc_side(mail_for_user, user_h, u_times, u_time_enc,
                        u_time_enc_k, gu_W1, gu_b1, gu_W2, gu_b2)
    return (user_new, item_new)
